# baseline (device time: 89977 ns/iter reference)
import functools

import jax
import jax.numpy as jnp
from jax import lax
from jax.experimental import pallas as pl
from jax.experimental.pallas import tpu as pltpu

N_DEV = 32
N_TOK = 512
D_IN = 256
D_OUT = 512
E_TOTAL = 128
E_LOCAL = 4
CAP = 3
S_PAD = 16

_HIGH = jax.lax.Precision.HIGHEST


def kernel(x, router_W, route_idx, expert_W):
    del router_W

    def body(x_ref, idx_ref, w_ref, out_ref, comm_ref, send_sems, recv_sems):
        my = lax.axis_index("i")
        right = lax.rem(my + 1, N_DEV)

        ids_col = idx_ref[:, :]

        e_iota = lax.broadcasted_iota(jnp.int32, (N_TOK, E_TOTAL), 1)
        onehot = (ids_col == e_iota).astype(jnp.float32)
        row_i = lax.broadcasted_iota(jnp.int32, (N_TOK, N_TOK), 0)
        col_j = lax.broadcasted_iota(jnp.int32, (N_TOK, N_TOK), 1)
        lstrict = (col_j < row_i).astype(jnp.float32)
        counts = jax.lax.dot_general(
            lstrict, onehot, (((1,), (0,)), ((), ())), precision=_HIGH
        )
        rank_col = jnp.sum(counts * onehot, axis=1, keepdims=True)

        q_j = col_j
        k = q_j // S_PAD
        s = q_j % S_PAD
        e_loc = s // 4
        c = s % 4
        shard = jnp.remainder(my - k, N_DEV)
        expert = shard * E_LOCAL + e_loc
        gt = (
            (ids_col == expert)
            & (rank_col == c.astype(jnp.float32))
            & (c < CAP)
        ).astype(jnp.float32)

        gt_my = gt[:, 0:S_PAD]
        gathered = jax.lax.dot_general(
            gt_my, x_ref[:, :], (((0,), (0,)), ((), ())), precision=_HIGH
        )
        y = jax.lax.dot_general(
            gathered.reshape(E_LOCAL, 4, D_IN),
            w_ref[:, :, :],
            (((2,), (1,)), ((0,), (0,))),
            precision=_HIGH,
        )
        comm_ref[0, :, :] = y.reshape(S_PAD, D_OUT)

        for h in range(N_DEV - 1):
            rdma = pltpu.make_async_remote_copy(
                src_ref=comm_ref.at[h],
                dst_ref=comm_ref.at[h + 1],
                send_sem=send_sems.at[h],
                recv_sem=recv_sems.at[h],
                device_id=(right,),
                device_id_type=pl.DeviceIdType.MESH,
            )
            rdma.start()
            rdma.wait()

        comm_flat = comm_ref[:, :, :].reshape(N_DEV * S_PAD, D_OUT)
        out_ref[:, :] = jax.lax.dot_general(
            gt, comm_flat, (((1,), (0,)), ((), ())), precision=_HIGH
        )

    return pl.pallas_call(
        body,
        out_shape=jax.ShapeDtypeStruct((N_TOK, D_OUT), jnp.float32),
        in_specs=[
            pl.BlockSpec(memory_space=pltpu.VMEM),
            pl.BlockSpec(memory_space=pltpu.VMEM),
            pl.BlockSpec(memory_space=pltpu.VMEM),
        ],
        out_specs=pl.BlockSpec(memory_space=pltpu.VMEM),
        scratch_shapes=[
            pltpu.VMEM((N_DEV, S_PAD, D_OUT), jnp.float32),
            pltpu.SemaphoreType.DMA((N_DEV - 1,)),
            pltpu.SemaphoreType.DMA((N_DEV - 1,)),
        ],
    )(x, route_idx, expert_W)


# device time: 35510 ns/iter; 2.5338x vs baseline; 2.5338x over previous
import functools

import jax
import jax.numpy as jnp
from jax import lax
from jax.experimental import pallas as pl
from jax.experimental.pallas import tpu as pltpu

N_DEV = 32
N_TOK = 512
D_IN = 256
D_OUT = 512
E_TOTAL = 128
E_LOCAL = 4
CAP = 3
S_PAD = 16

_HIGH = jax.lax.Precision.HIGHEST


def kernel(x, router_W, route_idx, expert_W):
    del router_W

    def body(x_ref, idx_ref, w_ref, out_ref, comm_ref, send_sems, recv_sems):
        my = lax.axis_index("i")

        ids_col = idx_ref[:, :]

        e_iota = lax.broadcasted_iota(jnp.int32, (N_TOK, E_TOTAL), 1)
        onehot = (ids_col == e_iota).astype(jnp.float32)
        row_i = lax.broadcasted_iota(jnp.int32, (N_TOK, N_TOK), 0)
        col_j = lax.broadcasted_iota(jnp.int32, (N_TOK, N_TOK), 1)
        lstrict = (col_j < row_i).astype(jnp.float32)
        counts = jax.lax.dot_general(
            lstrict, onehot, (((1,), (0,)), ((), ())), precision=_HIGH
        )
        rank_col = jnp.sum(counts * onehot, axis=1, keepdims=True)

        q_j = col_j
        k = q_j // S_PAD
        s = q_j % S_PAD
        e_loc = s // 4
        c = s % 4
        shard = jnp.remainder(my - k, N_DEV)
        expert = shard * E_LOCAL + e_loc
        gt = (
            (ids_col == expert)
            & (rank_col == c.astype(jnp.float32))
            & (c < CAP)
        ).astype(jnp.float32)

        gt_my = gt[:, 0:S_PAD]
        gathered = jax.lax.dot_general(
            gt_my, x_ref[:, :], (((0,), (0,)), ((), ())), precision=_HIGH
        )
        y = jax.lax.dot_general(
            gathered.reshape(E_LOCAL, 4, D_IN),
            w_ref[:, :, :],
            (((2,), (1,)), ((0,), (0,))),
            precision=_HIGH,
        )
        comm_ref[0, :, :] = y.reshape(S_PAD, D_OUT)

        sends = []
        for o in range(1, N_DEV):
            rdma = pltpu.make_async_remote_copy(
                src_ref=comm_ref.at[0],
                dst_ref=comm_ref.at[o],
                send_sem=send_sems.at[o - 1],
                recv_sem=recv_sems.at[o - 1],
                device_id=(lax.rem(my + o, N_DEV),),
                device_id_type=pl.DeviceIdType.MESH,
            )
            rdma.start()
            sends.append(rdma)
        for rdma in sends:
            rdma.wait_recv()
        for rdma in sends:
            rdma.wait_send()

        comm_flat = comm_ref[:, :, :].reshape(N_DEV * S_PAD, D_OUT)
        out_ref[:, :] = jax.lax.dot_general(
            gt, comm_flat, (((1,), (0,)), ((), ())), precision=_HIGH
        )

    return pl.pallas_call(
        body,
        out_shape=jax.ShapeDtypeStruct((N_TOK, D_OUT), jnp.float32),
        in_specs=[
            pl.BlockSpec(memory_space=pltpu.VMEM),
            pl.BlockSpec(memory_space=pltpu.VMEM),
            pl.BlockSpec(memory_space=pltpu.VMEM),
        ],
        out_specs=pl.BlockSpec(memory_space=pltpu.VMEM),
        scratch_shapes=[
            pltpu.VMEM((N_DEV, S_PAD, D_OUT), jnp.float32),
            pltpu.SemaphoreType.DMA((N_DEV - 1,)),
            pltpu.SemaphoreType.DMA((N_DEV - 1,)),
        ],
    )(x, route_idx, expert_W)


# device time: 20723 ns/iter; 4.3419x vs baseline; 1.7136x over previous
import jax
import jax.numpy as jnp
from jax import lax
from jax.experimental import pallas as pl
from jax.experimental.pallas import tpu as pltpu

N_DEV = 32
N_TOK = 512
D_IN = 256
D_OUT = 512
E_TOTAL = 128
E_LOCAL = 4
CAP = 3
S_PAD = 16


def kernel(x, router_W, route_idx, expert_W):
    del router_W

    def body(x_ref, idx_ref, w_ref, out_ref, comm_ref, send_sems, recv_sems):
        my = lax.axis_index("i")

        barrier_sem = pltpu.get_barrier_semaphore()

        def signal_one(o, _):
            pl.semaphore_signal(
                barrier_sem,
                inc=1,
                device_id=(lax.rem(my + o, N_DEV),),
                device_id_type=pl.DeviceIdType.MESH,
            )
            return 0

        lax.fori_loop(1, N_DEV, signal_one, 0)

        ids_col = idx_ref[:, :]

        e_iota = lax.broadcasted_iota(jnp.int32, (N_TOK, E_TOTAL), 1)
        onehot = (ids_col == e_iota).astype(jnp.float32)
        row_i = lax.broadcasted_iota(jnp.int32, (N_TOK, N_TOK), 0)
        col_j = lax.broadcasted_iota(jnp.int32, (N_TOK, N_TOK), 1)
        lstrict = (col_j < row_i).astype(jnp.float32)
        counts = jax.lax.dot_general(
            lstrict, onehot, (((1,), (0,)), ((), ()))
        )
        rank_col = jnp.sum(counts * onehot, axis=1, keepdims=True)

        def build_gt(n_slots):
            q_j = lax.broadcasted_iota(jnp.int32, (N_TOK, n_slots), 1)
            k = q_j // S_PAD
            s = q_j % S_PAD
            e_loc = s // 4
            c = s % 4
            shard = jnp.remainder(my - k, N_DEV)
            expert = shard * E_LOCAL + e_loc
            return (
                (ids_col == expert)
                & (rank_col == c.astype(jnp.float32))
                & (c < CAP)
            )

        gt_my = build_gt(S_PAD).astype(jnp.float32)
        gathered = jax.lax.dot_general(
            gt_my, x_ref[:, :], (((0,), (0,)), ((), ()))
        )
        y = jax.lax.dot_general(
            gathered.reshape(E_LOCAL, 4, D_IN),
            w_ref[:, :, :],
            (((2,), (1,)), ((0,), (0,))),
        )
        comm_ref[0, :, :] = y.reshape(S_PAD, D_OUT).astype(jnp.bfloat16)

        def peer_rdma(o):
            return pltpu.make_async_remote_copy(
                src_ref=comm_ref.at[0],
                dst_ref=comm_ref.at[o],
                send_sem=send_sems.at[o - 1],
                recv_sem=recv_sems.at[o - 1],
                device_id=(lax.rem(my + o, N_DEV),),
                device_id_type=pl.DeviceIdType.MESH,
            )

        pl.semaphore_wait(barrier_sem, N_DEV - 1)
        lax.fori_loop(1, N_DEV, lambda o, _: (peer_rdma(o).start(), 0)[1], 0)

        gt = build_gt(N_DEV * S_PAD).astype(jnp.bfloat16)

        lax.fori_loop(1, N_DEV, lambda o, _: (peer_rdma(o).wait_recv(), 0)[1], 0)
        lax.fori_loop(1, N_DEV, lambda o, _: (peer_rdma(o).wait_send(), 0)[1], 0)

        comm_flat = comm_ref[:, :, :].reshape(N_DEV * S_PAD, D_OUT)
        out_ref[:, :] = jax.lax.dot_general(
            gt,
            comm_flat,
            (((1,), (0,)), ((), ())),
            preferred_element_type=jnp.float32,
        )

    return pl.pallas_call(
        body,
        out_shape=jax.ShapeDtypeStruct((N_TOK, D_OUT), jnp.float32),
        in_specs=[
            pl.BlockSpec(memory_space=pltpu.VMEM),
            pl.BlockSpec(memory_space=pltpu.VMEM),
            pl.BlockSpec(memory_space=pltpu.VMEM),
        ],
        out_specs=pl.BlockSpec(memory_space=pltpu.VMEM),
        scratch_shapes=[
            pltpu.VMEM((N_DEV, S_PAD, D_OUT), jnp.bfloat16),
            pltpu.SemaphoreType.DMA((N_DEV - 1,)),
            pltpu.SemaphoreType.DMA((N_DEV - 1,)),
        ],
        compiler_params=pltpu.CompilerParams(collective_id=0),
    )(x, route_idx, expert_W)
